# Initial kernel scaffold; baseline (speedup 1.0000x reference)
import jax
import jax.numpy as jnp
from jax.experimental import pallas as pl


def kernel(user_ids, movie_ids, gender, age, occupation, genres, W_wide, b_wide, user_table, movie_table, W1, b1, W2, b2, W3, b3, W4, b4):
    raise NotImplementedError("write your pallas kernel here")


# speedup vs baseline: 19.4198x; 19.4198x over previous
import functools

import jax
import jax.numpy as jnp
from jax import lax
from jax.experimental import pallas as pl
from jax.experimental.pallas import tpu as pltpu
from jax.experimental.pallas import tpu_sc as plsc

NUM_USERS = 100000
NUM_MOVIES = 100000
EMBED_DIM = 32
NUM_GENRES = 18
BATCH = 1024

NC, NS, LANES = 2, 16, 16
NW = NC * NS
BPW = BATCH // NW
WIDE_ROWS = (NUM_USERS + NUM_MOVIES + NUM_GENRES + 3 + LANES - 1) // LANES


@functools.partial(
    pl.kernel,
    mesh=plsc.VectorSubcoreMesh(core_axis_name="c", subcore_axis_name="s"),
    compiler_params=pltpu.CompilerParams(
        use_tc_tiling_on_sc=False, needs_layout_passes=False),
    out_type=[
        jax.ShapeDtypeStruct((BATCH, EMBED_DIM), jnp.float32),
        jax.ShapeDtypeStruct((BATCH, EMBED_DIM), jnp.float32),
        jax.ShapeDtypeStruct((BATCH,), jnp.float32),
        jax.ShapeDtypeStruct((BATCH,), jnp.float32),
    ],
    scratch_types=[
        pltpu.VMEM((BPW,), jnp.int32),
        pltpu.VMEM((BPW,), jnp.int32),
        pltpu.VMEM((BPW,), jnp.int32),
        pltpu.VMEM((BPW,), jnp.int32),
        pltpu.VMEM((BPW, EMBED_DIM), jnp.float32),
        pltpu.VMEM((BPW, EMBED_DIM), jnp.float32),
        pltpu.VMEM((BPW, LANES), jnp.float32),
        pltpu.VMEM((BPW, LANES), jnp.float32),
        pltpu.VMEM((BPW,), jnp.float32),
        pltpu.VMEM((BPW,), jnp.float32),
        pltpu.SemaphoreType.DMA,
    ],
)
def _sc_gather(uid_hbm, mid_hbm, wpad_hbm, utab_hbm, mtab_hbm,
               uemb_out, memb_out, wu_out, wm_out,
               uidx_v, midx_v, urow_v, mrow_v, urows_v, mrows_v,
               wurows_v, wmrows_v, wu_v, wm_v, sem):
    wid = lax.axis_index("s") * NC + lax.axis_index("c")
    base = wid * BPW
    pltpu.sync_copy(uid_hbm.at[pl.ds(base, BPW)], uidx_v)
    pltpu.sync_copy(mid_hbm.at[pl.ds(base, BPW)], midx_v)
    for i in range(BPW // LANES):
        s = pl.ds(i * LANES, LANES)
        urow_v[s] = uidx_v[s] >> 4
        mrow_v[s] = (midx_v[s] + NUM_USERS) >> 4
    cp1 = pltpu.async_copy(utab_hbm.at[uidx_v], urows_v, sem)
    cp2 = pltpu.async_copy(mtab_hbm.at[midx_v], mrows_v, sem)
    cp3 = pltpu.async_copy(wpad_hbm.at[urow_v], wurows_v, sem)
    cp4 = pltpu.async_copy(wpad_hbm.at[mrow_v], wmrows_v, sem)
    cp1.wait()
    cp2.wait()
    cp3.wait()
    cp4.wait()
    for i in range(BPW // LANES):
        s = pl.ds(i * LANES, LANES)
        rows = jnp.arange(LANES, dtype=jnp.int32) + i * LANES
        wu_v[s] = plsc.load_gather(wurows_v, [rows, uidx_v[s] & 15])
        wm_v[s] = plsc.load_gather(wmrows_v, [rows, (midx_v[s] + NUM_USERS) & 15])
    pltpu.sync_copy(urows_v, uemb_out.at[pl.ds(base, BPW)])
    pltpu.sync_copy(mrows_v, memb_out.at[pl.ds(base, BPW)])
    pltpu.sync_copy(wu_v, wu_out.at[pl.ds(base, BPW)])
    pltpu.sync_copy(wm_v, wm_out.at[pl.ds(base, BPW)])


def _tc_mlp(uemb, memb, dense, wu, wm,
            w1u, w1m, w1d, b1, w2, b2, w3, b3, w4t, b4, wdt, bw, out_ref):
    d = dense[...]
    h = uemb[...] @ w1u[...] + memb[...] @ w1m[...] + d @ w1d[...] + b1[...]
    h = jnp.maximum(h, 0.0)
    h = jnp.maximum(h @ w2[...] + b2[...], 0.0)
    h = jnp.maximum(h @ w3[...] + b3[...], 0.0)
    deep = jnp.sum(h * w4t[...], axis=1, keepdims=True) + b4[...]
    wide = wu[...] + wm[...] + jnp.sum(d * wdt[...], axis=1, keepdims=True) + bw[...]
    x = wide + deep
    out_ref[...] = 1.0 / (1.0 + jnp.exp(-x))


def kernel(user_ids, movie_ids, gender, age, occupation, genres, W_wide,
           b_wide, user_table, movie_table, W1, b1, W2, b2, W3, b3, W4, b4):
    wpad = jnp.pad(W_wide[:, 0], (0, WIDE_ROWS * LANES - W_wide.shape[0])
                   ).reshape(WIDE_ROWS, LANES)
    uemb, memb, wu, wm = _sc_gather(
        user_ids, movie_ids, wpad, user_table, movie_table)
    dense = jnp.concatenate(
        [gender[:, None], age[:, None], occupation[:, None], genres], axis=1)
    out = pl.pallas_call(
        _tc_mlp,
        out_shape=jax.ShapeDtypeStruct((BATCH, 1), jnp.float32),
    )(uemb, memb, dense, wu.reshape(BATCH, 1), wm.reshape(BATCH, 1),
      W1[:EMBED_DIM], W1[EMBED_DIM:2 * EMBED_DIM], W1[2 * EMBED_DIM:],
      b1.reshape(1, -1), W2, b2.reshape(1, -1), W3, b3.reshape(1, -1),
      W4.reshape(1, -1), b4.reshape(1, 1),
      W_wide[NUM_USERS + NUM_MOVIES:, 0].reshape(1, -1), b_wide.reshape(1, 1))
    return out[:, 0]


# speedup vs baseline: 19.8477x; 1.0220x over previous
import functools

import jax
import jax.numpy as jnp
from jax import lax
from jax.experimental import pallas as pl
from jax.experimental.pallas import tpu as pltpu
from jax.experimental.pallas import tpu_sc as plsc

NUM_USERS = 100000
NUM_MOVIES = 100000
EMBED_DIM = 32
NUM_GENRES = 18
BATCH = 1024

NC, NS, LANES = 2, 16, 16
NW = NC * NS
BPW = BATCH // NW
ROWW = 128
PER_ROW = ROWW // EMBED_DIM
WIDE_ONEHOT = NUM_USERS + NUM_MOVIES
WIDE_ROWS = (WIDE_ONEHOT + ROWW - 1) // ROWW


@functools.partial(
    pl.kernel,
    mesh=plsc.VectorSubcoreMesh(core_axis_name="c", subcore_axis_name="s"),
    compiler_params=pltpu.CompilerParams(needs_layout_passes=False),
    out_type=[
        jax.ShapeDtypeStruct((BATCH, ROWW), jnp.float32),
        jax.ShapeDtypeStruct((BATCH, ROWW), jnp.float32),
        jax.ShapeDtypeStruct((BATCH,), jnp.float32),
    ],
    scratch_types=[
        pltpu.VMEM((BPW,), jnp.int32),
        pltpu.VMEM((BPW,), jnp.int32),
        pltpu.VMEM((BPW,), jnp.int32),
        pltpu.VMEM((BPW,), jnp.int32),
        pltpu.VMEM((BPW, ROWW), jnp.float32),
        pltpu.VMEM((BPW, ROWW), jnp.float32),
        pltpu.VMEM((BPW, ROWW), jnp.float32),
        pltpu.VMEM((BPW, ROWW), jnp.float32),
        pltpu.VMEM((BPW,), jnp.float32),
        pltpu.SemaphoreType.DMA,
    ],
)
def _sc_gather(uid_hbm, mid_hbm, wpad_hbm, utab_hbm, mtab_hbm,
               urows_out, mrows_out, wide_out,
               uidx_v, midx_v, urow_v, mrow_v, urows_v, mrows_v,
               wurows_v, wmrows_v, wide_v, sem):
    wid = lax.axis_index("s") * NC + lax.axis_index("c")
    base = wid * BPW
    pltpu.sync_copy(uid_hbm.at[pl.ds(base, BPW)], uidx_v)
    pltpu.sync_copy(mid_hbm.at[pl.ds(base, BPW)], midx_v)
    for i in range(BPW // LANES):
        s = pl.ds(i * LANES, LANES)
        urow_v[s] = uidx_v[s] >> 2
        mrow_v[s] = midx_v[s] >> 2
    cp1 = pltpu.async_copy(utab_hbm.at[urow_v], urows_v, sem)
    cp2 = pltpu.async_copy(mtab_hbm.at[mrow_v], mrows_v, sem)
    cp1.wait()
    cp2.wait()
    pltpu.sync_copy(urows_v, urows_out.at[pl.ds(base, BPW)])
    pltpu.sync_copy(mrows_v, mrows_out.at[pl.ds(base, BPW)])
    for i in range(BPW // LANES):
        s = pl.ds(i * LANES, LANES)
        urow_v[s] = uidx_v[s] >> 7
        mrow_v[s] = (midx_v[s] + NUM_USERS) >> 7
    cp3 = pltpu.async_copy(wpad_hbm.at[urow_v], wurows_v, sem)
    cp4 = pltpu.async_copy(wpad_hbm.at[mrow_v], wmrows_v, sem)
    cp3.wait()
    cp4.wait()
    for i in range(BPW // LANES):
        s = pl.ds(i * LANES, LANES)
        rows = jnp.arange(LANES, dtype=jnp.int32) + i * LANES
        wu = plsc.load_gather(wurows_v, [rows, uidx_v[s] & (ROWW - 1)])
        wm = plsc.load_gather(
            wmrows_v, [rows, (midx_v[s] + NUM_USERS) & (ROWW - 1)])
        wide_v[s] = wu + wm
    pltpu.sync_copy(wide_v, wide_out.at[pl.ds(base, BPW)])


def _pick32(rows128, ids):
    lane = lax.broadcasted_iota(jnp.int32, (BATCH, ROWW), 1)
    sel = jnp.where((lane >> 5) == (ids & 3), rows128, 0.0)
    return (sel[:, 0:32] + sel[:, 32:64] + sel[:, 64:96] + sel[:, 96:128])


def _tc_mlp(urows, mrows, uid, mid, dense, wg,
            w1u, w1m, w1d, b1, w2, b2, w3, b3, w4t, b4, wdt, bw, out_ref):
    d = dense[...]
    uemb = _pick32(urows[...], uid[...])
    memb = _pick32(mrows[...], mid[...])
    h = uemb @ w1u[...] + memb @ w1m[...] + d @ w1d[...] + b1[...]
    h = jnp.maximum(h, 0.0)
    h = jnp.maximum(h @ w2[...] + b2[...], 0.0)
    h = jnp.maximum(h @ w3[...] + b3[...], 0.0)
    deep = jnp.sum(h * w4t[...], axis=1, keepdims=True) + b4[...]
    wide = wg[...] + jnp.sum(d * wdt[...], axis=1, keepdims=True) + bw[...]
    x = wide + deep
    out_ref[...] = 1.0 / (1.0 + jnp.exp(-x))


def kernel(user_ids, movie_ids, gender, age, occupation, genres, W_wide,
           b_wide, user_table, movie_table, W1, b1, W2, b2, W3, b3, W4, b4):
    ut128 = user_table.reshape(NUM_USERS * EMBED_DIM // ROWW, ROWW)
    mt128 = movie_table.reshape(NUM_MOVIES * EMBED_DIM // ROWW, ROWW)
    wpad = jnp.pad(W_wide[:WIDE_ONEHOT, 0],
                   (0, WIDE_ROWS * ROWW - WIDE_ONEHOT)).reshape(WIDE_ROWS, ROWW)
    urows, mrows, wide = _sc_gather(user_ids, movie_ids, wpad, ut128, mt128)
    dense = jnp.concatenate(
        [gender[:, None], age[:, None], occupation[:, None], genres], axis=1)
    out = pl.pallas_call(
        _tc_mlp,
        out_shape=jax.ShapeDtypeStruct((BATCH, 1), jnp.float32),
    )(urows, mrows, user_ids.reshape(BATCH, 1), movie_ids.reshape(BATCH, 1),
      dense, wide.reshape(BATCH, 1),
      W1[:EMBED_DIM], W1[EMBED_DIM:2 * EMBED_DIM], W1[2 * EMBED_DIM:],
      b1.reshape(1, -1), W2, b2.reshape(1, -1), W3, b3.reshape(1, -1),
      W4.reshape(1, -1), b4.reshape(1, 1),
      W_wide[WIDE_ONEHOT:, 0].reshape(1, -1), b_wide.reshape(1, 1))
    return out[:, 0]
